# Initial kernel scaffold; baseline (speedup 1.0000x reference)
import functools

import jax
import jax.numpy as jnp
from jax import lax
from jax.experimental import pallas as pl
from jax.experimental.pallas import tpu as pltpu
from jax.experimental.pallas import tpu_sc as plsc

N_NODES = 10000
N_EDGES = 320000
D = 128
BN_EPS = 1e-5

NC = 2
NS = 16
NW = NC * NS
EDGES_PER_WORKER = N_EDGES // NW
WIN = 400
N_INIT_WORKERS = 10
ROWS_PER_INIT = N_NODES // N_INIT_WORKERS


def _sc_aggregate(h, src, dst, zeros):
    mesh = plsc.VectorSubcoreMesh(core_axis_name="c", subcore_axis_name="s")
    out_t = jax.ShapeDtypeStruct((N_NODES, D), jnp.float32)

    @functools.partial(
        pl.kernel,
        mesh=mesh,
        out_type=[out_t, out_t],
        scratch_types=[
            pltpu.VMEM_SHARED((N_NODES, D), jnp.float32),
            pltpu.VMEM((WIN,), jnp.int32),
            pltpu.VMEM((WIN,), jnp.int32),
            pltpu.VMEM((WIN, D), jnp.float32),
            pltpu.SemaphoreType.DMA,
        ],
    )
    def k(h_hbm, src_hbm, dst_hbm, z_hbm, out0, out1, acc, sidx, didx, rows, sem):
        c = lax.axis_index("c")
        s = lax.axis_index("s")

        @pl.when(s < N_INIT_WORKERS)
        def _():
            r0 = s * ROWS_PER_INIT

            @pl.when(c == 0)
            def _():
                pltpu.async_copy(
                    h_hbm.at[pl.ds(r0, ROWS_PER_INIT)],
                    acc.at[pl.ds(r0, ROWS_PER_INIT)], sem).wait()

            @pl.when(c != 0)
            def _():
                pltpu.async_copy(
                    z_hbm.at[pl.ds(r0, ROWS_PER_INIT)],
                    acc.at[pl.ds(r0, ROWS_PER_INIT)], sem).wait()

        plsc.subcore_barrier()

        wid = s * NC + c
        e_base = wid * EDGES_PER_WORKER

        @pl.loop(0, EDGES_PER_WORKER, step=WIN)
        def _(off):
            b = e_base + off
            pltpu.sync_copy(src_hbm.at[pl.ds(b, WIN)], sidx)
            pltpu.sync_copy(dst_hbm.at[pl.ds(b, WIN)], didx)
            pltpu.async_copy(h_hbm.at[sidx], rows, sem).wait()
            pltpu.sync_copy(rows, acc.at[didx], add=True)

        plsc.subcore_barrier()

        @pl.when(s < N_INIT_WORKERS)
        def _():
            r0 = s * ROWS_PER_INIT

            @pl.when(c == 0)
            def _():
                pltpu.async_copy(
                    acc.at[pl.ds(r0, ROWS_PER_INIT)],
                    out0.at[pl.ds(r0, ROWS_PER_INIT)], sem).wait()

            @pl.when(c != 0)
            def _():
                pltpu.async_copy(
                    acc.at[pl.ds(r0, ROWS_PER_INIT)],
                    out1.at[pl.ds(r0, ROWS_PER_INIT)], sem).wait()

    return k(h, src, dst, zeros)


def _mlp_body(a0, a1, w1, b1, g, bt, w2, b2, out):
    u = a0[...] + a1[...]
    y = jnp.dot(u, w1[...], preferred_element_type=jnp.float32,
                precision=lax.Precision.HIGHEST) + b1[...]
    mu = jnp.mean(y, axis=0, keepdims=True)
    d = y - mu
    var = jnp.mean(d * d, axis=0, keepdims=True)
    yn = d * (g[...] * lax.rsqrt(var + BN_EPS)) + bt[...]
    yn = jnp.maximum(yn, 0.0)
    z = jnp.dot(yn, w2[...], preferred_element_type=jnp.float32,
                precision=lax.Precision.HIGHEST) + b2[...]
    out[...] = jnp.maximum(z, 0.0)


_mlp = pl.pallas_call(
    _mlp_body,
    out_shape=jax.ShapeDtypeStruct((N_NODES, D), jnp.float32),
)


def kernel(x, edge_index, params):
    src = edge_index[0]
    dst = edge_index[1]
    zeros = jnp.zeros((N_NODES, D), jnp.float32)
    h = x
    for (W1, b1, gamma, beta, W2, b2) in params:
        p0, p1 = _sc_aggregate(h, src, dst, zeros)
        h = _mlp(p0, p1, W1, b1.reshape(1, D), gamma.reshape(1, D),
                 beta.reshape(1, D), W2, b2.reshape(1, D))
    return h


# speedup vs baseline: 6.8389x; 6.8389x over previous
import functools

import jax
import jax.numpy as jnp
from jax import lax
from jax.experimental import pallas as pl
from jax.experimental.pallas import tpu as pltpu
from jax.experimental.pallas import tpu_sc as plsc

N_NODES = 10000
N_EDGES = 320000
D = 128
BN_EPS = 1e-5

NC = 2
NS = 16
NW = NC * NS
EDGES_PER_WORKER = N_EDGES // NW
WIN = 200
N_INIT_WORKERS = 10
ROWS_PER_INIT = N_NODES // N_INIT_WORKERS


def _sc_aggregate(h, src, dst, zeros):
    mesh = plsc.VectorSubcoreMesh(core_axis_name="c", subcore_axis_name="s")
    out_t = jax.ShapeDtypeStruct((N_NODES, D), jnp.float32)

    @functools.partial(
        pl.kernel,
        mesh=mesh,
        out_type=[out_t, out_t],
        scratch_types=[
            pltpu.VMEM_SHARED((N_NODES, D), jnp.float32),
            pltpu.VMEM((WIN,), jnp.int32),
            pltpu.VMEM((WIN,), jnp.int32),
            pltpu.VMEM((WIN, D), jnp.float32),
            pltpu.SemaphoreType.DMA,
        ],
    )
    def k(h_hbm, src_hbm, dst_hbm, z_hbm, out0, out1, acc, sidx, didx, rows, sem):
        c = lax.axis_index("c")
        s = lax.axis_index("s")

        @pl.when(s < N_INIT_WORKERS)
        def _():
            r0 = s * ROWS_PER_INIT

            @pl.when(c == 0)
            def _():
                pltpu.async_copy(
                    h_hbm.at[pl.ds(r0, ROWS_PER_INIT)],
                    acc.at[pl.ds(r0, ROWS_PER_INIT)], sem).wait()

            @pl.when(c != 0)
            def _():
                pltpu.async_copy(
                    z_hbm.at[pl.ds(r0, ROWS_PER_INIT)],
                    acc.at[pl.ds(r0, ROWS_PER_INIT)], sem).wait()

        plsc.subcore_barrier()

        wid = s * NC + c
        e_base = wid * EDGES_PER_WORKER

        @pl.loop(0, EDGES_PER_WORKER, step=WIN)
        def _(off):
            b = e_base + off
            pltpu.sync_copy(src_hbm.at[pl.ds(b, WIN)], sidx)
            pltpu.sync_copy(dst_hbm.at[pl.ds(b, WIN)], didx)
            pltpu.async_copy(h_hbm.at[sidx], rows, sem).wait()
            pltpu.sync_copy(rows, acc.at[didx], add=True)

        plsc.subcore_barrier()

        @pl.when(s < N_INIT_WORKERS)
        def _():
            r0 = s * ROWS_PER_INIT

            @pl.when(c == 0)
            def _():
                pltpu.async_copy(
                    acc.at[pl.ds(r0, ROWS_PER_INIT)],
                    out0.at[pl.ds(r0, ROWS_PER_INIT)], sem).wait()

            @pl.when(c != 0)
            def _():
                pltpu.async_copy(
                    acc.at[pl.ds(r0, ROWS_PER_INIT)],
                    out1.at[pl.ds(r0, ROWS_PER_INIT)], sem).wait()

    return k(h, src, dst, zeros)


def _mlp_body(a0, a1, w1, b1, g, bt, w2, b2, out):
    u = a0[...] + a1[...]
    y = jnp.dot(u, w1[...], preferred_element_type=jnp.float32,
                precision=lax.Precision.DEFAULT) + b1[...]
    mu = jnp.mean(y, axis=0, keepdims=True)
    d = y - mu
    var = jnp.mean(d * d, axis=0, keepdims=True)
    yn = d * (g[...] * lax.rsqrt(var + BN_EPS)) + bt[...]
    yn = jnp.maximum(yn, 0.0)
    z = jnp.dot(yn, w2[...], preferred_element_type=jnp.float32,
                precision=lax.Precision.DEFAULT) + b2[...]
    out[...] = jnp.maximum(z, 0.0)


_mlp = pl.pallas_call(
    _mlp_body,
    out_shape=jax.ShapeDtypeStruct((N_NODES, D), jnp.float32),
)


def kernel(x, edge_index, params):
    src = edge_index[0]
    dst = edge_index[1]
    zeros = jnp.zeros((N_NODES, D), jnp.float32)
    h = x
    for (W1, b1, gamma, beta, W2, b2) in params:
        p0, p1 = _sc_aggregate(h, src, dst, zeros)
        h = _mlp(p0, p1, W1, b1.reshape(1, D), gamma.reshape(1, D),
                 beta.reshape(1, D), W2, b2.reshape(1, D))
    return h


# speedup vs baseline: 10.6953x; 1.5639x over previous
import functools

import jax
import jax.numpy as jnp
from jax import lax
from jax.experimental import pallas as pl
from jax.experimental.pallas import tpu as pltpu
from jax.experimental.pallas import tpu_sc as plsc

N_NODES = 10000
N_EDGES = 320000
D = 128
BN_EPS = 1e-5

NC = 2
NS = 16
NW = NC * NS
EDGES_PER_WORKER = N_EDGES // NW
WIN = 80
NWIN = EDGES_PER_WORKER // WIN
N_INIT_WORKERS = 10
ROWS_PER_INIT = N_NODES // N_INIT_WORKERS


def _sc_aggregate(h, src, dst, zeros):
    mesh = plsc.VectorSubcoreMesh(core_axis_name="c", subcore_axis_name="s")
    out_t = jax.ShapeDtypeStruct((N_NODES, D), jnp.float32)

    @functools.partial(
        pl.kernel,
        mesh=mesh,
        out_type=[out_t, out_t],
        scratch_types=[
            pltpu.VMEM_SHARED((N_NODES, D), jnp.float32),
            pltpu.VMEM((NWIN, WIN), jnp.int32),
            pltpu.VMEM((EDGES_PER_WORKER,), jnp.int32),
            pltpu.VMEM((WIN, D), jnp.float32),
            pltpu.VMEM((WIN, D), jnp.float32),
            pltpu.SemaphoreType.DMA,
            pltpu.SemaphoreType.DMA,
            pltpu.SemaphoreType.DMA,
            pltpu.SemaphoreType.DMA,
        ],
    )
    def k(h_hbm, src_hbm, dst_hbm, z_hbm, out0, out1,
          acc, didx, sidx, rows0, rows1,
          gsem0, gsem1, ssem0, ssem1):
        c = lax.axis_index("c")
        s = lax.axis_index("s")

        @pl.when(s < N_INIT_WORKERS)
        def _():
            r0 = s * ROWS_PER_INIT

            @pl.when(c == 0)
            def _():
                pltpu.async_copy(
                    h_hbm.at[pl.ds(r0, ROWS_PER_INIT)],
                    acc.at[pl.ds(r0, ROWS_PER_INIT)], gsem0).wait()

            @pl.when(c != 0)
            def _():
                pltpu.async_copy(
                    z_hbm.at[pl.ds(r0, ROWS_PER_INIT)],
                    acc.at[pl.ds(r0, ROWS_PER_INIT)], gsem0).wait()

        plsc.subcore_barrier()

        wid = s * NC + c

        pltpu.sync_copy(dst_hbm.at[wid], didx)
        pltpu.sync_copy(src_hbm.at[wid], sidx)

        pltpu.make_async_copy(
            h_hbm.at[sidx.at[pl.ds(0, WIN)]], rows0, gsem0).start()

        @pl.loop(0, NWIN - 1, step=2)
        def _(g):
            @pl.when(g > 0)
            def _():
                pltpu.make_async_copy(rows1, acc.at[didx.at[0]], ssem1).wait()

            pltpu.make_async_copy(
                h_hbm.at[sidx.at[pl.ds((g + 1) * WIN, WIN)]],
                rows1, gsem1).start()
            pltpu.make_async_copy(
                h_hbm.at[sidx.at[pl.ds(g * WIN, WIN)]], rows0, gsem0).wait()
            pltpu.async_copy(rows0, acc.at[didx.at[g]], ssem0, add=True)

            pltpu.make_async_copy(rows0, acc.at[didx.at[0]], ssem0).wait()
            pltpu.make_async_copy(
                h_hbm.at[sidx.at[pl.ds((g + 2) * WIN, WIN)]],
                rows0, gsem0).start()
            pltpu.make_async_copy(
                h_hbm.at[sidx.at[pl.ds((g + 1) * WIN, WIN)]],
                rows1, gsem1).wait()
            pltpu.async_copy(rows1, acc.at[didx.at[g + 1]], ssem1, add=True)

        pltpu.make_async_copy(
            h_hbm.at[sidx.at[pl.ds((NWIN - 1) * WIN, WIN)]],
            rows0, gsem0).wait()
        pltpu.async_copy(rows0, acc.at[didx.at[NWIN - 1]], ssem0, add=True)
        pltpu.make_async_copy(rows1, acc.at[didx.at[0]], ssem1).wait()
        pltpu.make_async_copy(rows0, acc.at[didx.at[0]], ssem0).wait()

        plsc.subcore_barrier()

        @pl.when(s < N_INIT_WORKERS)
        def _():
            r0 = s * ROWS_PER_INIT

            @pl.when(c == 0)
            def _():
                pltpu.async_copy(
                    acc.at[pl.ds(r0, ROWS_PER_INIT)],
                    out0.at[pl.ds(r0, ROWS_PER_INIT)], gsem0).wait()

            @pl.when(c != 0)
            def _():
                pltpu.async_copy(
                    acc.at[pl.ds(r0, ROWS_PER_INIT)],
                    out1.at[pl.ds(r0, ROWS_PER_INIT)], gsem0).wait()

    return k(h, src, dst, zeros)


def _mlp_body(a0, a1, w1, b1, g, bt, w2, b2, out):
    u = a0[...] + a1[...]
    y = jnp.dot(u, w1[...], preferred_element_type=jnp.float32,
                precision=lax.Precision.DEFAULT) + b1[...]
    mu = jnp.mean(y, axis=0, keepdims=True)
    d = y - mu
    var = jnp.mean(d * d, axis=0, keepdims=True)
    yn = d * (g[...] * lax.rsqrt(var + BN_EPS)) + bt[...]
    yn = jnp.maximum(yn, 0.0)
    z = jnp.dot(yn, w2[...], preferred_element_type=jnp.float32,
                precision=lax.Precision.DEFAULT) + b2[...]
    out[...] = jnp.maximum(z, 0.0)


_mlp = pl.pallas_call(
    _mlp_body,
    out_shape=jax.ShapeDtypeStruct((N_NODES, D), jnp.float32),
)


def kernel(x, edge_index, params):
    src = edge_index[0].reshape(NW, EDGES_PER_WORKER)
    dst = edge_index[1].reshape(NW, NWIN, WIN)
    zeros = jnp.zeros((N_NODES, D), jnp.float32)
    h = x
    for (W1, b1, gamma, beta, W2, b2) in params:
        p0, p1 = _sc_aggregate(h, src, dst, zeros)
        h = _mlp(p0, p1, W1, b1.reshape(1, D), gamma.reshape(1, D),
                 beta.reshape(1, D), W2, b2.reshape(1, D))
    return h


# speedup vs baseline: 10.8304x; 1.0126x over previous
import functools

import jax
import jax.numpy as jnp
from jax import lax
from jax.experimental import pallas as pl
from jax.experimental.pallas import tpu as pltpu
from jax.experimental.pallas import tpu_sc as plsc

N_NODES = 10000
N_EDGES = 320000
D = 128
BN_EPS = 1e-5

NC = 2
NS = 16
NW = NC * NS
EDGES_PER_WORKER = N_EDGES // NW
WIN = 80
NWIN = EDGES_PER_WORKER // WIN
INIT_ROWS = 624
INIT_ROWS_LAST = N_NODES - 15 * INIT_ROWS


def _sc_aggregate(h, src, dst, zeros):
    mesh = plsc.VectorSubcoreMesh(core_axis_name="c", subcore_axis_name="s")
    out_t = jax.ShapeDtypeStruct((N_NODES, D), jnp.float32)

    @functools.partial(
        pl.kernel,
        mesh=mesh,
        out_type=[out_t, out_t],
        scratch_types=[
            pltpu.VMEM_SHARED((N_NODES, D), jnp.float32),
            pltpu.VMEM((NWIN, WIN), jnp.int32),
            pltpu.VMEM((EDGES_PER_WORKER,), jnp.int32),
            pltpu.VMEM((WIN, D), jnp.float32),
            pltpu.VMEM((WIN, D), jnp.float32),
            pltpu.SemaphoreType.DMA,
            pltpu.SemaphoreType.DMA,
            pltpu.SemaphoreType.DMA,
            pltpu.SemaphoreType.DMA,
        ],
    )
    def k(h_hbm, src_hbm, dst_hbm, z_hbm, out0, out1,
          acc, didx, sidx, rows0, rows1,
          gsem0, gsem1, ssem0, ssem1):
        c = lax.axis_index("c")
        s = lax.axis_index("s")

        wid = s * NC + c

        r0 = s * INIT_ROWS
        nr = jnp.where(s == NS - 1, INIT_ROWS_LAST, INIT_ROWS)

        @pl.when(c == 0)
        def _():
            pltpu.async_copy(
                h_hbm.at[pl.ds(r0, nr)], acc.at[pl.ds(r0, nr)], gsem0)

        @pl.when(c != 0)
        def _():
            pltpu.async_copy(
                z_hbm.at[pl.ds(r0, nr)], acc.at[pl.ds(r0, nr)], gsem0)

        pltpu.sync_copy(dst_hbm.at[wid], didx)
        pltpu.sync_copy(src_hbm.at[wid], sidx)

        pltpu.make_async_copy(
            h_hbm.at[pl.ds(r0, nr)], acc.at[pl.ds(r0, nr)], gsem0).wait()

        plsc.subcore_barrier()

        HALF = WIN // 2

        def gather_start(w, rows, sem):
            b = w * WIN
            pltpu.make_async_copy(
                h_hbm.at[sidx.at[pl.ds(b, HALF)]],
                rows.at[pl.ds(0, HALF)], sem).start()
            pltpu.make_async_copy(
                h_hbm.at[sidx.at[pl.ds(b + HALF, HALF)]],
                rows.at[pl.ds(HALF, HALF)], sem).start()

        def gather_wait(rows, sem):
            pltpu.make_async_copy(
                h_hbm.at[sidx.at[pl.ds(0, HALF)]],
                rows.at[pl.ds(0, HALF)], sem).wait()
            pltpu.make_async_copy(
                h_hbm.at[sidx.at[pl.ds(0, HALF)]],
                rows.at[pl.ds(HALF, HALF)], sem).wait()

        gather_start(0, rows0, gsem0)

        @pl.loop(0, NWIN - 1, step=2)
        def _(g):
            @pl.when(g > 0)
            def _():
                pltpu.make_async_copy(rows1, acc.at[didx.at[0]], ssem1).wait()

            gather_start(g + 1, rows1, gsem1)
            gather_wait(rows0, gsem0)
            pltpu.async_copy(rows0, acc.at[didx.at[g]], ssem0, add=True)

            pltpu.make_async_copy(rows0, acc.at[didx.at[0]], ssem0).wait()
            gather_start(g + 2, rows0, gsem0)
            gather_wait(rows1, gsem1)
            pltpu.async_copy(rows1, acc.at[didx.at[g + 1]], ssem1, add=True)

        gather_wait(rows0, gsem0)
        pltpu.async_copy(rows0, acc.at[didx.at[NWIN - 1]], ssem0, add=True)
        pltpu.make_async_copy(rows1, acc.at[didx.at[0]], ssem1).wait()
        pltpu.make_async_copy(rows0, acc.at[didx.at[0]], ssem0).wait()

        plsc.subcore_barrier()

        @pl.when(c == 0)
        def _():
            pltpu.async_copy(
                acc.at[pl.ds(r0, nr)], out0.at[pl.ds(r0, nr)], gsem0).wait()

        @pl.when(c != 0)
        def _():
            pltpu.async_copy(
                acc.at[pl.ds(r0, nr)], out1.at[pl.ds(r0, nr)], gsem0).wait()

    return k(h, src, dst, zeros)


def _mlp_body(a0, a1, w1, b1, g, bt, w2, b2, out):
    u = a0[...] + a1[...]
    y = jnp.dot(u, w1[...], preferred_element_type=jnp.float32,
                precision=lax.Precision.DEFAULT) + b1[...]
    mu = jnp.mean(y, axis=0, keepdims=True)
    d = y - mu
    var = jnp.mean(d * d, axis=0, keepdims=True)
    yn = d * (g[...] * lax.rsqrt(var + BN_EPS)) + bt[...]
    yn = jnp.maximum(yn, 0.0)
    z = jnp.dot(yn, w2[...], preferred_element_type=jnp.float32,
                precision=lax.Precision.DEFAULT) + b2[...]
    out[...] = jnp.maximum(z, 0.0)


_mlp = pl.pallas_call(
    _mlp_body,
    out_shape=jax.ShapeDtypeStruct((N_NODES, D), jnp.float32),
)


def kernel(x, edge_index, params):
    src = edge_index[0].reshape(NW, EDGES_PER_WORKER)
    dst = edge_index[1].reshape(NW, NWIN, WIN)
    zeros = jnp.zeros((N_NODES, D), jnp.float32)
    h = x
    for (W1, b1, gamma, beta, W2, b2) in params:
        p0, p1 = _sc_aggregate(h, src, dst, zeros)
        h = _mlp(p0, p1, W1, b1.reshape(1, D), gamma.reshape(1, D),
                 beta.reshape(1, D), W2, b2.reshape(1, D))
    return h


# speedup vs baseline: 10.8405x; 1.0009x over previous
import functools

import jax
import jax.numpy as jnp
from jax import lax
from jax.experimental import pallas as pl
from jax.experimental.pallas import tpu as pltpu
from jax.experimental.pallas import tpu_sc as plsc

N_NODES = 10000
N_EDGES = 320000
D = 128
BN_EPS = 1e-5

NC = 2
NS = 16
NW = NC * NS
EDGES_PER_WORKER = N_EDGES // NW
WIN = 80
NWIN = EDGES_PER_WORKER // WIN
N_INIT_WORKERS = 10
ROWS_PER_INIT = N_NODES // N_INIT_WORKERS


def _sc_aggregate(h, src, dst, zeros):
    mesh = plsc.VectorSubcoreMesh(core_axis_name="c", subcore_axis_name="s")
    out_t = jax.ShapeDtypeStruct((N_NODES, D), jnp.float32)

    @functools.partial(
        pl.kernel,
        mesh=mesh,
        out_type=[out_t, out_t],
        scratch_types=[
            pltpu.VMEM_SHARED((N_NODES, D), jnp.float32),
            pltpu.VMEM((NWIN, WIN), jnp.int32),
            pltpu.VMEM((EDGES_PER_WORKER,), jnp.int32),
            pltpu.VMEM((WIN, D), jnp.float32),
            pltpu.VMEM((WIN, D), jnp.float32),
            pltpu.SemaphoreType.DMA,
            pltpu.SemaphoreType.DMA,
            pltpu.SemaphoreType.DMA,
            pltpu.SemaphoreType.DMA,
        ],
    )
    def k(h_hbm, src_hbm, dst_hbm, z_hbm, out0, out1,
          acc, didx, sidx, rows0, rows1,
          gsem0, gsem1, ssem0, ssem1):
        c = lax.axis_index("c")
        s = lax.axis_index("s")

        wid = s * NC + c

        @pl.when(s < N_INIT_WORKERS)
        def _():
            r0 = s * ROWS_PER_INIT

            @pl.when(c == 0)
            def _():
                pltpu.async_copy(
                    h_hbm.at[pl.ds(r0, ROWS_PER_INIT)],
                    acc.at[pl.ds(r0, ROWS_PER_INIT)], gsem0)

            @pl.when(c != 0)
            def _():
                pltpu.async_copy(
                    z_hbm.at[pl.ds(r0, ROWS_PER_INIT)],
                    acc.at[pl.ds(r0, ROWS_PER_INIT)], gsem0)

        pltpu.sync_copy(dst_hbm.at[wid], didx)
        pltpu.sync_copy(src_hbm.at[wid], sidx)

        @pl.when(s < N_INIT_WORKERS)
        def _():
            r0 = s * ROWS_PER_INIT
            pltpu.make_async_copy(
                h_hbm.at[pl.ds(r0, ROWS_PER_INIT)],
                acc.at[pl.ds(r0, ROWS_PER_INIT)], gsem0).wait()

        plsc.subcore_barrier()

        HALF = WIN // 2

        def gather_start(w, rows, sem):
            b = w * WIN
            pltpu.make_async_copy(
                h_hbm.at[sidx.at[pl.ds(b, HALF)]],
                rows.at[pl.ds(0, HALF)], sem).start()
            pltpu.make_async_copy(
                h_hbm.at[sidx.at[pl.ds(b + HALF, HALF)]],
                rows.at[pl.ds(HALF, HALF)], sem).start()

        def gather_wait(rows, sem):
            pltpu.make_async_copy(
                h_hbm.at[sidx.at[pl.ds(0, HALF)]],
                rows.at[pl.ds(0, HALF)], sem).wait()
            pltpu.make_async_copy(
                h_hbm.at[sidx.at[pl.ds(0, HALF)]],
                rows.at[pl.ds(HALF, HALF)], sem).wait()

        gather_start(0, rows0, gsem0)

        @pl.loop(0, NWIN - 1, step=2)
        def _(g):
            @pl.when(g > 0)
            def _():
                pltpu.make_async_copy(rows1, acc.at[didx.at[0]], ssem1).wait()

            gather_start(g + 1, rows1, gsem1)
            gather_wait(rows0, gsem0)
            pltpu.async_copy(rows0, acc.at[didx.at[g]], ssem0, add=True)

            pltpu.make_async_copy(rows0, acc.at[didx.at[0]], ssem0).wait()
            gather_start(g + 2, rows0, gsem0)
            gather_wait(rows1, gsem1)
            pltpu.async_copy(rows1, acc.at[didx.at[g + 1]], ssem1, add=True)

        gather_wait(rows0, gsem0)
        pltpu.async_copy(rows0, acc.at[didx.at[NWIN - 1]], ssem0, add=True)
        pltpu.make_async_copy(rows1, acc.at[didx.at[0]], ssem1).wait()
        pltpu.make_async_copy(rows0, acc.at[didx.at[0]], ssem0).wait()

        plsc.subcore_barrier()

        @pl.when(s < N_INIT_WORKERS)
        def _():
            r0 = s * ROWS_PER_INIT

            @pl.when(c == 0)
            def _():
                pltpu.async_copy(
                    acc.at[pl.ds(r0, ROWS_PER_INIT)],
                    out0.at[pl.ds(r0, ROWS_PER_INIT)], gsem0).wait()

            @pl.when(c != 0)
            def _():
                pltpu.async_copy(
                    acc.at[pl.ds(r0, ROWS_PER_INIT)],
                    out1.at[pl.ds(r0, ROWS_PER_INIT)], gsem0).wait()

    return k(h, src, dst, zeros)


def _mlp_body(a0, a1, w1, b1, g, bt, w2, b2, out):
    u = a0[...] + a1[...]
    y = jnp.dot(u, w1[...], preferred_element_type=jnp.float32,
                precision=lax.Precision.DEFAULT) + b1[...]
    mu = jnp.mean(y, axis=0, keepdims=True)
    d = y - mu
    var = jnp.mean(d * d, axis=0, keepdims=True)
    yn = d * (g[...] * lax.rsqrt(var + BN_EPS)) + bt[...]
    yn = jnp.maximum(yn, 0.0)
    z = jnp.dot(yn, w2[...], preferred_element_type=jnp.float32,
                precision=lax.Precision.DEFAULT) + b2[...]
    out[...] = jnp.maximum(z, 0.0)


_mlp = pl.pallas_call(
    _mlp_body,
    out_shape=jax.ShapeDtypeStruct((N_NODES, D), jnp.float32),
)


def kernel(x, edge_index, params):
    src = edge_index[0].reshape(NW, EDGES_PER_WORKER)
    dst = edge_index[1].reshape(NW, NWIN, WIN)
    zeros = jnp.zeros((N_NODES, D), jnp.float32)
    h = x
    for (W1, b1, gamma, beta, W2, b2) in params:
        p0, p1 = _sc_aggregate(h, src, dst, zeros)
        h = _mlp(p0, p1, W1, b1.reshape(1, D), gamma.reshape(1, D),
                 beta.reshape(1, D), W2, b2.reshape(1, D))
    return h
